# baseline (speedup 1.0000x reference)
import functools

import jax
import jax.numpy as jnp
import numpy as np
from jax import lax
from jax.experimental import pallas as pl
from jax.experimental.pallas import tpu as pltpu
from jax.experimental.pallas import tpu_sc as plsc

N = 10000
E = 160000
NUM_SPECIES = 10
F = 128
NB = 8
SH = 9
R_MAX = 5.0
EPS = 0.5
HR = 64
HRO = 16

NC, NS, L = 2, 16, 16
NW = NC * NS
CHUNK = 64
SLOTS = 160
NPASS = SLOTS // NW
EPT = E // NW
NVR = EPT // L + 1
PERM_LEN = E + SLOTS * L + 128
N_PAD = SLOTS * CHUNK
W_EDGE = 128
AF = SH * F
EPS_SQ = EPS * EPS

_mesh = plsc.VectorSubcoreMesh(core_axis_name="c", subcore_axis_name="s")


def _wid():
    return lax.axis_index("s") * NC + lax.axis_index("c")


def _iota():
    return lax.iota(jnp.int32, L)


def _extract(vec, j):
    return jnp.sum(jnp.where(_iota() == j, vec, 0))


_GDN = lax.GatherDimensionNumbers(
    offset_dims=(), collapsed_slice_dims=(0,), start_index_map=(0,))


def _lane_bcast(vec, j):
    idx = jnp.full((L, 1), j, jnp.int32)
    return lax.gather(vec, idx, _GDN, slice_sizes=(1,),
                      mode=lax.GatherScatterMode.PROMISE_IN_BOUNDS)


@functools.partial(
    pl.kernel, mesh=_mesh,
    compiler_params=pltpu.CompilerParams(needs_layout_passes=False),
    out_type=jax.ShapeDtypeStruct((NW * SLOTS * L,), jnp.int32),
    scratch_types=[
        pltpu.VMEM((EPT + L,), jnp.int32),
        pltpu.VMEM((SLOTS * L,), jnp.int32),
    ],
)
def _hist_kernel(rcv_hbm, hist_hbm, rcv_v, hist_v):
    w = _wid()

    def zero(i, _):
        hist_v[pl.ds(i * L, L)] = jnp.zeros((L,), jnp.int32)
        return 0
    lax.fori_loop(0, SLOTS, zero, 0)

    pltpu.sync_copy(rcv_hbm.at[pl.ds(w * EPT, EPT)], rcv_v.at[pl.ds(0, EPT)])

    def body(i, _):
        m = i * L + _iota() < EPT
        r = jnp.where(m, rcv_v[pl.ds(i * L, L)], 0)
        cidx = lax.shift_right_logical(r, 6) * L + _iota()
        cur = plsc.load_gather(hist_v, [cidx])
        plsc.store_scatter(hist_v, [cidx], cur + 1, mask=m)
        return 0
    lax.fori_loop(0, NVR, body, 0)

    pltpu.sync_copy(hist_v, hist_hbm.at[pl.ds(w * SLOTS * L, SLOTS * L)])


@functools.partial(
    pl.kernel, mesh=_mesh,
    compiler_params=pltpu.CompilerParams(needs_layout_passes=False),
    out_type=(
        jax.ShapeDtypeStruct((NW * SLOTS * L,), jnp.int32),
        jax.ShapeDtypeStruct((SLOTS * L,), jnp.int32),
        jax.ShapeDtypeStruct((SLOTS * L,), jnp.int32),
    ),
    scratch_types=[
        pltpu.VMEM((NW * SLOTS * L,), jnp.int32),
        pltpu.VMEM((SLOTS * L,), jnp.int32),
        pltpu.VMEM((NW, NPASS * L), jnp.int32),
        pltpu.VMEM((NPASS * L,), jnp.int32),
        pltpu.VMEM((NPASS * L,), jnp.int32),
    ],
)
def _scan_kernel(hist_hbm, off_hbm, bstart_hbm, bcnt_hbm,
                 hist_v, colsum_v, off_v, bst_v, bct_v):
    w = _wid()
    b0 = w * NPASS
    pltpu.sync_copy(hist_hbm, hist_v)

    def csum(c, _):
        def acc(t, a):
            return a + hist_v[pl.ds(t * SLOTS * L + c * L, L)]
        v = lax.fori_loop(0, NW, acc, jnp.zeros((L,), jnp.int32))
        colsum_v[pl.ds(c * L, L)] = v
        return 0
    lax.fori_loop(0, SLOTS, csum, 0)

    def pb(b, base):
        tot = jnp.sum(colsum_v[pl.ds(b * L, L)])
        return base + ((tot + L - 1) // L) * L
    pbase = lax.fori_loop(0, b0, pb, jnp.int32(0))

    def bucket(p, pbase):
        b = b0 + p
        tot = jnp.sum(colsum_v[pl.ds(b * L, L)])
        bst_v[pl.ds(p * L, L)] = jnp.full((L,), pbase, jnp.int32)
        bct_v[pl.ds(p * L, L)] = jnp.full((L,), tot, jnp.int32)

        def tile(t, carry):
            h = hist_v[pl.ds(t * SLOTS * L + b * L, L)]
            incl = plsc.cumsum(h)
            off_v[t, pl.ds(p * L, L)] = pbase + carry + incl - h
            return carry + jnp.sum(h)
        lax.fori_loop(0, NW, tile, jnp.int32(0))
        return pbase + ((tot + L - 1) // L) * L
    lax.fori_loop(0, NPASS, bucket, pbase)

    def wr(t, _):
        pltpu.sync_copy(
            off_v.at[t],
            off_hbm.at[pl.ds(t * SLOTS * L + b0 * L, NPASS * L)])
        return 0
    lax.fori_loop(0, NW, wr, 0)
    pltpu.sync_copy(bst_v, bstart_hbm.at[pl.ds(b0 * L, NPASS * L)])
    pltpu.sync_copy(bct_v, bcnt_hbm.at[pl.ds(b0 * L, NPASS * L)])


NCHUNK = (EPT + 127) // 128 + 1


@functools.partial(
    pl.kernel, mesh=_mesh,
    compiler_params=pltpu.CompilerParams(needs_layout_passes=False),
    out_type=jax.ShapeDtypeStruct((PERM_LEN,), jnp.int32),
    scratch_types=[
        pltpu.VMEM((EPT + L,), jnp.int32),
        pltpu.VMEM((SLOTS * L,), jnp.int32),
        pltpu.VMEM((NCHUNK, 128), jnp.int32),
        pltpu.VMEM((NCHUNK, 128), jnp.int32),
        pltpu.SemaphoreType.DMA,
    ],
)
def _permute_kernel(rcv_hbm, off_hbm, perm_hbm, rcv_v, cnt_v, dst_v, val_v, sem):
    w = _wid()
    pltpu.sync_copy(rcv_hbm.at[pl.ds(w * EPT, EPT)], rcv_v.at[pl.ds(0, EPT)])
    pltpu.sync_copy(off_hbm.at[pl.ds(w * SLOTS * L, SLOTS * L)], cnt_v)

    def body(i, _):
        pos = i * L + _iota()
        m = pos < EPT
        r = jnp.where(m, rcv_v[pl.ds(i * L, L)], 0)
        cidx = lax.shift_right_logical(r, 6) * L + _iota()
        d = plsc.load_gather(cnt_v, [cidx])
        plsc.store_scatter(cnt_v, [cidx], d + 1, mask=m)
        d = jnp.where(m, d, PERM_LEN - 128 + _iota())
        eid = w * EPT + pos
        packed = jnp.where(m, eid | lax.shift_left(r & (CHUNK - 1), 18), 0)
        c = i >> 3
        j = (i & 7) * L
        dst_v[c, pl.ds(j, L)] = d
        val_v[c, pl.ds(j, L)] = packed
        return 0
    lax.fori_loop(0, NVR, body, 0)
    def padtail(i, _):
        c = i >> 3
        j = (i & 7) * L
        dst_v[c, pl.ds(j, L)] = PERM_LEN - 128 + _iota()
        val_v[c, pl.ds(j, L)] = jnp.zeros((L,), jnp.int32)
        return 0
    lax.fori_loop(NVR, NCHUNK * 8, padtail, 0)

    def scat(c, _):
        pltpu.async_copy(val_v.at[c], perm_hbm.at[dst_v.at[c]], sem).wait()
        return 0
    lax.fori_loop(0, NCHUNK, scat, 0)


@functools.partial(
    pl.kernel, mesh=_mesh,
    compiler_params=pltpu.CompilerParams(needs_layout_passes=False),
    out_type=(
        jax.ShapeDtypeStruct((N_PAD, F), jnp.float32),
        jax.ShapeDtypeStruct((N_PAD, F), jnp.float32),
    ),
    scratch_types=[
        pltpu.VMEM((CHUNK * AF,), jnp.float32),
        pltpu.VMEM((W_EDGE,), jnp.int32),
        pltpu.VMEM((W_EDGE,), jnp.int32),
        pltpu.VMEM((W_EDGE,), jnp.int32),
        pltpu.VMEM((W_EDGE,), jnp.int32),
        pltpu.VMEM((W_EDGE,), jnp.int32),
        pltpu.VMEM((W_EDGE, 8 * L), jnp.float32),
        pltpu.VMEM((W_EDGE, F), jnp.float32),
        pltpu.VMEM((CHUNK, F), jnp.float32),
        pltpu.VMEM((CHUNK, F), jnp.float32),
        pltpu.VMEM((L,), jnp.int32),
        pltpu.VMEM((L,), jnp.int32),
        pltpu.SemaphoreType.DMA,
        pltpu.SemaphoreType.DMA,
        pltpu.SemaphoreType.DMA,
    ],
)
def _edge_kernel(perm_hbm, bstart_hbm, bcnt_hbm, snd_hbm, yr_hbm, s_hbm,
                 a0_hbm, scal_hbm,
                 A_v, raw_v, ids_v, rl_v, snd_v, grp_v, yr_v, s_v, a0o_v, sco_v,
                 bst_v, bct_v, sem1, sem2, sem3):
    w = _wid()

    def per_bucket(p, _):
        b = w * NPASS + p

        def zero(i, _):
            A_v[pl.ds(i * L, L)] = jnp.zeros((L,), jnp.float32)
            return 0
        lax.fori_loop(0, CHUNK * AF // L, zero, 0)

        pltpu.sync_copy(bstart_hbm.at[pl.ds(b * L, L)], bst_v)
        pltpu.sync_copy(bcnt_hbm.at[pl.ds(b * L, L)], bct_v)
        start = pl.multiple_of(jnp.max(bst_v[...]), L)
        cnt = jnp.max(bct_v[...])
        nwin = (cnt + W_EDGE - 1) // W_EDGE

        def window(wi, _):
            woff = wi * W_EDGE
            pltpu.sync_copy(perm_hbm.at[pl.ds(start + woff, W_EDGE)], raw_v)

            def san(j, _):
                pos = woff + j * L + _iota()
                m = pos < cnt
                raw = jnp.where(m, raw_v[pl.ds(j * L, L)], 0)
                eid = raw & 0x3FFFF
                ids_v[pl.ds(j * L, L)] = eid
                grp_v[pl.ds(j * L, L)] = lax.shift_right_logical(eid, 3)
                rl_v[pl.ds(j * L, L)] = lax.shift_right_logical(raw, 18)
                return 0
            lax.fori_loop(0, W_EDGE // L, san, 0)

            cp_yr = pltpu.async_copy(yr_hbm.at[grp_v], yr_v, sem1)
            cp_sn = pltpu.async_copy(snd_hbm.at[ids_v], snd_v, sem2)
            cp_sn.wait()
            cp_s = pltpu.async_copy(s_hbm.at[snd_v], s_v, sem3)
            cp_yr.wait()
            cp_s.wait()

            trip = jnp.minimum(W_EDGE, cnt - woff)

            def edge(e, _):
                vi = lax.shift_right_logical(e, 4) * L
                lane = e & (L - 1)
                rl = _extract(rl_v[pl.ds(vi, L)], lane)
                abase = rl * AF
                sub = _extract(ids_v[pl.ds(vi, L)], lane) & 7
                yr = yr_v[e, pl.ds(sub * L, L)]
                for a in range(SH):
                    ya = _lane_bcast(yr, a)
                    for k in range(F // L):
                        addr = abase + a * F + k * L
                        A_v[pl.ds(addr, L)] += ya * s_v[e, pl.ds(k * L, L)]
                return 0
            lax.fori_loop(0, trip, edge, 0)
            return 0
        lax.fori_loop(0, nwin, window, 0)

        def node(n, _):
            for k in range(F // L):
                acc = jnp.zeros((L,), jnp.float32)
                for a in range(SH):
                    v = A_v[pl.ds(n * AF + a * F + k * L, L)]
                    acc = acc + v * v
                sco_v[n, pl.ds(k * L, L)] = acc * EPS_SQ
                a0o_v[n, pl.ds(k * L, L)] = A_v[pl.ds(n * AF + k * L, L)] * EPS
            return 0
        lax.fori_loop(0, CHUNK, node, 0)

        nb = pl.multiple_of(b * CHUNK, CHUNK)
        pltpu.sync_copy(a0o_v, a0_hbm.at[pl.ds(nb, CHUNK)])
        pltpu.sync_copy(sco_v, scal_hbm.at[pl.ds(nb, CHUNK)])
        return 0
    lax.fori_loop(0, NPASS, per_bucket, 0)


NODE_BLK = 512


def _node_phase_body(a0_ref, sc_ref, cw_ref, wlin_ref, out_ref):
    scal = sc_ref[...]
    cw = cw_ref[...]
    g = cw[:, 0:F] + cw[:, F:2 * F] * scal + cw[:, 2 * F:3 * F] * (scal * scal)
    b0 = a0_ref[...] * g
    out_ref[...] = jnp.dot(b0, wlin_ref[...], preferred_element_type=jnp.float32)


def _node_phase(a0, scal, cw, Wlin):
    return pl.pallas_call(
        _node_phase_body,
        grid=(N_PAD // NODE_BLK,),
        in_specs=[
            pl.BlockSpec((NODE_BLK, F), lambda i: (i, 0)),
            pl.BlockSpec((NODE_BLK, F), lambda i: (i, 0)),
            pl.BlockSpec((NODE_BLK, 3 * F), lambda i: (i, 0)),
            pl.BlockSpec((F, F), lambda i: (0, 0)),
        ],
        out_specs=pl.BlockSpec((NODE_BLK, F), lambda i: (i, 0)),
        out_shape=jax.ShapeDtypeStruct((N_PAD, F), jnp.float32),
    )(a0, scal, cw, Wlin)


def _sph(u):
    x, y, z = u[:, 0], u[:, 1], u[:, 2]
    s3 = float(np.sqrt(3.0)); s15 = float(np.sqrt(15.0)); s5 = float(np.sqrt(5.0))
    comps = [jnp.ones_like(x), s3 * x, s3 * y, s3 * z,
             s15 * x * y, s15 * y * z, 0.5 * s5 * (3.0 * z * z - 1.0),
             s15 * x * z, 0.5 * s15 * (x * x - y * y)]
    return jnp.stack(comps, axis=-1)


def _radial(r):
    n = jnp.arange(1, NB + 1, dtype=jnp.float32)
    rs = jnp.clip(r, 1e-9, None)
    rb = np.sqrt(2.0 / R_MAX) * jnp.sin(n * jnp.pi * rs / R_MAX) / rs
    x = r / R_MAX
    env = 1.0 - 21.0 * x ** 5 + 35.0 * x ** 6 - 15.0 * x ** 7
    env = jnp.where(x < 1.0, env, 0.0)
    return rb * env


def kernel(vectors, node_specie, senders, receivers, W_embed, W_up0, Wr1_0, Wr2_0, Wc0, Wlin0, Wro0, W_up1, Wr1_1, Wr2_1, Wc1, Wsc_lin1, Wsc_sp1, Wlin1, Wro1a, Wro1b):
    senders = senders.astype(jnp.int32)
    receivers = receivers.astype(jnp.int32)

    lengths = jnp.sqrt(jnp.sum(vectors * vectors, axis=-1, keepdims=True) + 1e-12)
    Y = _sph(vectors / lengths)
    ef = _radial(lengths)
    yr0 = Y * (jax.nn.silu(ef @ Wr1_0) @ Wr2_0)
    yr1 = Y * (jax.nn.silu(ef @ Wr1_1) @ Wr2_1)
    pad7 = jnp.zeros((E, L - SH), jnp.float32)
    yr0p = jnp.concatenate([yr0, pad7], axis=1).reshape(E // 8, 8 * L)
    yr1p = jnp.concatenate([yr1, pad7], axis=1).reshape(E // 8, 8 * L)

    emb = W_embed[node_specie]
    h0 = emb @ W_up0
    cw0 = Wc0[node_specie].reshape(N, 3 * F)
    cw1 = Wc1[node_specie].reshape(N, 3 * F)
    cw0p = jnp.concatenate([cw0, jnp.zeros((N_PAD - N, 3 * F), jnp.float32)], 0)
    cw1p = jnp.concatenate([cw1, jnp.zeros((N_PAD - N, 3 * F), jnp.float32)], 0)

    hist = _hist_kernel(receivers)
    offs, bstart, bcnt = _scan_kernel(hist)
    perm = _permute_kernel(receivers, offs)

    def interaction(h, yrp, cwp, Wlin):
        hp = jnp.concatenate([h, jnp.zeros((N_PAD - N, F), jnp.float32)], 0)
        a0, scal = _edge_kernel(perm, bstart, bcnt, senders, yrp, hp)
        return _node_phase(a0, scal, cwp, Wlin)[:N]

    nf1_0 = interaction(h0, yr0p, cw0p, Wlin0)
    ro0 = nf1_0 @ Wro0

    h1 = nf1_0 @ W_up1
    nf2_0 = interaction(h1, yr1p, cw1p, Wlin1)
    nf2_0 = nf2_0 + (nf1_0 @ Wsc_lin1) * Wsc_sp1[node_specie]
    ro1 = jax.nn.silu(nf2_0 @ Wro1a) @ Wro1b
    return jnp.stack([ro0, ro1], axis=1)


# speedup vs baseline: 2.1666x; 2.1666x over previous
import functools

import jax
import jax.numpy as jnp
import numpy as np
from jax import lax
from jax.experimental import pallas as pl
from jax.experimental.pallas import tpu as pltpu
from jax.experimental.pallas import tpu_sc as plsc

N = 10000
E = 160000
NUM_SPECIES = 10
F = 128
NB = 8
SH = 9
R_MAX = 5.0
EPS = 0.5
HR = 64
HRO = 16

NC, NS, L = 2, 16, 16
NW = NC * NS
CHUNK = 64
SLOTS = 160
NPASS = SLOTS // NW
EPT = E // NW
NVR = EPT // L + 1
PERM_LEN = 162816
N_PAD = SLOTS * CHUNK
W_EDGE = 64
AF = SH * F
EPS_SQ = EPS * EPS

_mesh = plsc.VectorSubcoreMesh(core_axis_name="c", subcore_axis_name="s")


def _wid():
    return lax.axis_index("s") * NC + lax.axis_index("c")


def _iota():
    return lax.iota(jnp.int32, L)


def _extract(vec, j):
    return jnp.sum(jnp.where(_iota() == j, vec, 0))


_GDN = lax.GatherDimensionNumbers(
    offset_dims=(), collapsed_slice_dims=(0,), start_index_map=(0,))


def _lane_bcast(vec, j):
    idx = jnp.full((L, 1), j, jnp.int32)
    return lax.gather(vec, idx, _GDN, slice_sizes=(1,),
                      mode=lax.GatherScatterMode.PROMISE_IN_BOUNDS)


@functools.partial(
    pl.kernel, mesh=_mesh,
    compiler_params=pltpu.CompilerParams(needs_layout_passes=False),
    out_type=jax.ShapeDtypeStruct((NW * SLOTS * L,), jnp.int32),
    scratch_types=[
        pltpu.VMEM((EPT + L,), jnp.int32),
        pltpu.VMEM((SLOTS * L,), jnp.int32),
    ],
)
def _hist_kernel(rcv_hbm, hist_hbm, rcv_v, hist_v):
    w = _wid()

    def zero(i, _):
        hist_v[pl.ds(i * L, L)] = jnp.zeros((L,), jnp.int32)
        return 0
    lax.fori_loop(0, SLOTS, zero, 0)

    pltpu.sync_copy(rcv_hbm.at[pl.ds(w * EPT, EPT)], rcv_v.at[pl.ds(0, EPT)])

    def body(i, _):
        m = i * L + _iota() < EPT
        r = jnp.where(m, rcv_v[pl.ds(i * L, L)], 0)
        cidx = lax.shift_right_logical(r, 6) * L + _iota()
        cur = plsc.load_gather(hist_v, [cidx])
        plsc.store_scatter(hist_v, [cidx], cur + 1, mask=m)
        return 0
    lax.fori_loop(0, NVR, body, 0)

    pltpu.sync_copy(hist_v, hist_hbm.at[pl.ds(w * SLOTS * L, SLOTS * L)])


@functools.partial(
    pl.kernel, mesh=_mesh,
    compiler_params=pltpu.CompilerParams(needs_layout_passes=False),
    out_type=(
        jax.ShapeDtypeStruct((NW * SLOTS * L,), jnp.int32),
        jax.ShapeDtypeStruct((SLOTS * L,), jnp.int32),
        jax.ShapeDtypeStruct((SLOTS * L,), jnp.int32),
    ),
    scratch_types=[
        pltpu.VMEM((NW * SLOTS * L,), jnp.int32),
        pltpu.VMEM((SLOTS * L,), jnp.int32),
        pltpu.VMEM((NW, NPASS * L), jnp.int32),
        pltpu.VMEM((NPASS * L,), jnp.int32),
        pltpu.VMEM((NPASS * L,), jnp.int32),
    ],
)
def _scan_kernel(hist_hbm, off_hbm, bstart_hbm, bcnt_hbm,
                 hist_v, colsum_v, off_v, bst_v, bct_v):
    w = _wid()
    b0 = w * NPASS
    pltpu.sync_copy(hist_hbm, hist_v)

    def csum(c, _):
        def acc(t, a):
            return a + hist_v[pl.ds(t * SLOTS * L + c * L, L)]
        v = lax.fori_loop(0, NW, acc, jnp.zeros((L,), jnp.int32))
        colsum_v[pl.ds(c * L, L)] = v
        return 0
    lax.fori_loop(0, SLOTS, csum, 0)

    def pb(b, base):
        tot = jnp.sum(colsum_v[pl.ds(b * L, L)])
        return base + ((tot + L - 1) // L) * L
    pbase = lax.fori_loop(0, b0, pb, jnp.int32(0))

    def bucket(p, pbase):
        b = b0 + p
        tot = jnp.sum(colsum_v[pl.ds(b * L, L)])
        bst_v[pl.ds(p * L, L)] = jnp.full((L,), pbase, jnp.int32)
        bct_v[pl.ds(p * L, L)] = jnp.full((L,), tot, jnp.int32)

        def tile(t, carry):
            h = hist_v[pl.ds(t * SLOTS * L + b * L, L)]
            incl = plsc.cumsum(h)
            off_v[t, pl.ds(p * L, L)] = pbase + carry + incl - h
            return carry + jnp.sum(h)
        lax.fori_loop(0, NW, tile, jnp.int32(0))
        return pbase + ((tot + L - 1) // L) * L
    lax.fori_loop(0, NPASS, bucket, pbase)

    def wr(t, _):
        pltpu.sync_copy(
            off_v.at[t],
            off_hbm.at[pl.ds(t * SLOTS * L + b0 * L, NPASS * L)])
        return 0
    lax.fori_loop(0, NW, wr, 0)
    pltpu.sync_copy(bst_v, bstart_hbm.at[pl.ds(b0 * L, NPASS * L)])
    pltpu.sync_copy(bct_v, bcnt_hbm.at[pl.ds(b0 * L, NPASS * L)])


NCHUNK = (EPT + 127) // 128 + 1


@functools.partial(
    pl.kernel, mesh=_mesh,
    compiler_params=pltpu.CompilerParams(needs_layout_passes=False),
    out_type=jax.ShapeDtypeStruct((2 * PERM_LEN,), jnp.int32),
    scratch_types=[
        pltpu.VMEM((EPT + L,), jnp.int32),
        pltpu.VMEM((EPT + L,), jnp.int32),
        pltpu.VMEM((SLOTS * L,), jnp.int32),
        pltpu.VMEM((NCHUNK, 128), jnp.int32),
        pltpu.VMEM((NCHUNK, 128), jnp.int32),
        pltpu.VMEM((NCHUNK, 128), jnp.int32),
        pltpu.SemaphoreType.DMA,
    ],
)
def _permute_kernel(rcv_hbm, snd_hbm, off_hbm, perm_hbm,
                    rcv_v, snd_v, cnt_v, dst_v, val_v, val2_v, sem):
    w = _wid()
    pltpu.sync_copy(rcv_hbm.at[pl.ds(w * EPT, EPT)], rcv_v.at[pl.ds(0, EPT)])
    pltpu.sync_copy(snd_hbm.at[pl.ds(w * EPT, EPT)], snd_v.at[pl.ds(0, EPT)])
    pltpu.sync_copy(off_hbm.at[pl.ds(w * SLOTS * L, SLOTS * L)], cnt_v)

    def body(i, _):
        pos = i * L + _iota()
        m = pos < EPT
        r = jnp.where(m, rcv_v[pl.ds(i * L, L)], 0)
        cidx = lax.shift_right_logical(r, 6) * L + _iota()
        d = plsc.load_gather(cnt_v, [cidx])
        plsc.store_scatter(cnt_v, [cidx], d + 1, mask=m)
        d = jnp.where(m, d, PERM_LEN - 128 + _iota())
        eid = w * EPT + pos
        packed = jnp.where(m, eid | lax.shift_left(r & (CHUNK - 1), 18), 0)
        c = i >> 3
        j = (i & 7) * L
        dst_v[c, pl.ds(j, L)] = d
        val_v[c, pl.ds(j, L)] = packed
        val2_v[c, pl.ds(j, L)] = jnp.where(m, snd_v[pl.ds(i * L, L)], 0)
        return 0
    lax.fori_loop(0, NVR, body, 0)

    def padtail(i, _):
        c = i >> 3
        j = (i & 7) * L
        dst_v[c, pl.ds(j, L)] = PERM_LEN - 128 + _iota()
        val_v[c, pl.ds(j, L)] = jnp.zeros((L,), jnp.int32)
        val2_v[c, pl.ds(j, L)] = jnp.zeros((L,), jnp.int32)
        return 0
    lax.fori_loop(NVR, NCHUNK * 8, padtail, 0)

    for g in range(0, NCHUNK, 4):
        cps = []
        for c in range(g, min(g + 4, NCHUNK)):
            cps.append(pltpu.async_copy(
                val_v.at[c], perm_hbm.at[dst_v.at[c]], sem))
            for j8 in range(8):
                dst_v[c, pl.ds(j8 * L, L)] = (
                    dst_v[c, pl.ds(j8 * L, L)] + PERM_LEN)
            cps.append(pltpu.async_copy(
                val2_v.at[c], perm_hbm.at[dst_v.at[c]], sem))
        for cp in cps:
            cp.wait()


@functools.partial(
    pl.kernel, mesh=_mesh,
    compiler_params=pltpu.CompilerParams(needs_layout_passes=False,
                                         use_tc_tiling_on_sc=False),
    out_type=jax.ShapeDtypeStruct((N_PAD, 2 * F), jnp.float32),
    scratch_types=[
        pltpu.VMEM((CHUNK * AF,), jnp.float32),
        pltpu.VMEM((2 * W_EDGE,), jnp.int32),
        pltpu.VMEM((2 * W_EDGE,), jnp.int32),
        pltpu.VMEM((2 * W_EDGE,), jnp.int32),
        pltpu.VMEM((2 * W_EDGE, 8 * L), jnp.float32),
        pltpu.VMEM((2 * W_EDGE, F), jnp.float32),
        pltpu.VMEM((L,), jnp.int32),
        pltpu.VMEM((L,), jnp.int32),
        pltpu.SemaphoreType.DMA,
        pltpu.SemaphoreType.DMA,
        pltpu.SemaphoreType.DMA,
        pltpu.SemaphoreType.DMA,
    ],
)
def _edge_kernel(perm_hbm, bstart_hbm, bcnt_hbm, yr_hbm, s_hbm,
                 out_hbm,
                 A_v, raw_v, snd_v, grp_v, yr_v, s_v, bst_v, bct_v,
                 sem_yr0, sem_yr1, sem_s0, sem_s1):
    w = _wid()

    def per_bucket(p, _):
        b = w * NPASS + p

        def zero(i, _):
            A_v[pl.ds(i * L, L)] = jnp.zeros((L,), jnp.float32)
            return 0
        lax.fori_loop(0, CHUNK * AF // L, zero, 0)

        pltpu.sync_copy(bstart_hbm.at[pl.ds(b * L, L)], bst_v)
        pltpu.sync_copy(bcnt_hbm.at[pl.ds(b * L, L)], bct_v)
        start = pl.multiple_of(jnp.max(bst_v[...]), L)
        cnt = jnp.max(bct_v[...])
        nwin = (cnt + W_EDGE - 1) // W_EDGE

        def prefetch(pw, par):
            po = par * W_EDGE
            sem_yr = sem_yr0 if par == 0 else sem_yr1
            sem_s = sem_s0 if par == 0 else sem_s1
            eoff = pl.multiple_of(start + pw * W_EDGE, L)
            pltpu.sync_copy(perm_hbm.at[pl.ds(eoff, W_EDGE)],
                            raw_v.at[pl.ds(po, W_EDGE)])
            pltpu.sync_copy(perm_hbm.at[pl.ds(PERM_LEN + eoff, W_EDGE)],
                            snd_v.at[pl.ds(po, W_EDGE)])

            def san(j, _):
                o = po + j * L
                pos = pw * W_EDGE + j * L + _iota()
                m = pos < cnt
                raw = jnp.where(m, raw_v[pl.ds(o, L)], 0)
                eid = raw & 0x3FFFF
                raw_v[pl.ds(o, L)] = raw
                grp_v[pl.ds(o, L)] = lax.shift_right_logical(eid, 3)
                snd_v[pl.ds(o, L)] = jnp.where(m, snd_v[pl.ds(o, L)], 0)
                return 0
            lax.fori_loop(0, W_EDGE // L, san, 0)

            pltpu.async_copy(yr_hbm.at[grp_v.at[pl.ds(po, W_EDGE)]],
                             yr_v.at[pl.ds(po, W_EDGE)], sem_yr)
            pltpu.async_copy(s_hbm.at[snd_v.at[pl.ds(po, W_EDGE)]],
                             s_v.at[pl.ds(po, W_EDGE)], sem_s)

        def wait_bufs(par):
            po = par * W_EDGE
            sem_yr = sem_yr0 if par == 0 else sem_yr1
            sem_s = sem_s0 if par == 0 else sem_s1
            pltpu.make_async_copy(yr_hbm.at[grp_v.at[pl.ds(po, W_EDGE)]],
                                  yr_v.at[pl.ds(po, W_EDGE)], sem_yr).wait()
            pltpu.make_async_copy(s_hbm.at[snd_v.at[pl.ds(po, W_EDGE)]],
                                  s_v.at[pl.ds(po, W_EDGE)], sem_s).wait()

        def compute(wc, par):
            po = par * W_EDGE
            trip = jnp.minimum(W_EDGE, cnt - wc * W_EDGE)

            def edge(e, _):
                vi = lax.shift_right_logical(e, 4) * L
                lane = e & (L - 1)
                raw = _lane_bcast(raw_v[pl.ds(po + vi, L)], lane)[0]
                rl = lax.shift_right_logical(raw, 18)
                sub = raw & 7
                abase = rl * AF
                yr = yr_v[po + e, pl.ds(sub * L, L)]
                for a in range(SH):
                    ya = _lane_bcast(yr, a)
                    for k in range(F // L):
                        addr = abase + a * F + k * L
                        plsc.addupdate(A_v.at[pl.ds(addr, L)],
                                       ya * s_v[po + e, pl.ds(k * L, L)])
                return 0
            lax.fori_loop(0, trip, edge, 0)

        @pl.when(nwin > 0)
        def _():
            prefetch(0, 0)

        def pair(i, _):
            w0 = 2 * i
            wait_bufs(0)

            @pl.when(w0 + 1 < nwin)
            def _():
                prefetch(w0 + 1, 1)
            compute(w0, 0)

            @pl.when(w0 + 1 < nwin)
            def _():
                wait_bufs(1)

                @pl.when(w0 + 2 < nwin)
                def _():
                    prefetch(w0 + 2, 0)
                compute(w0 + 1, 1)
            return 0
        lax.fori_loop(0, (nwin + 1) // 2, pair, 0)

        def node(n, _):
            for k in range(F // L):
                acc = jnp.zeros((L,), jnp.float32)
                for a in range(SH):
                    v = A_v[pl.ds(n * AF + a * F + k * L, L)]
                    acc = acc + v * v
                s_v[CHUNK + n, pl.ds(k * L, L)] = acc * EPS_SQ
                s_v[n, pl.ds(k * L, L)] = A_v[pl.ds(n * AF + k * L, L)] * EPS
            return 0
        lax.fori_loop(0, CHUNK, node, 0)

        nb = pl.multiple_of(b * CHUNK, CHUNK)
        pltpu.sync_copy(s_v.at[pl.ds(0, CHUNK)],
                        out_hbm.at[pl.ds(nb, CHUNK), pl.ds(0, F)])
        pltpu.sync_copy(s_v.at[pl.ds(CHUNK, CHUNK)],
                        out_hbm.at[pl.ds(nb, CHUNK), pl.ds(F, F)])
        return 0
    lax.fori_loop(0, NPASS, per_bucket, 0)


NODE_BLK = 512


def _node_phase_body(a0_ref, sc_ref, cw_ref, wlin_ref, out_ref):
    scal = sc_ref[...]
    cw = cw_ref[...]
    g = cw[:, 0:F] + cw[:, F:2 * F] * scal + cw[:, 2 * F:3 * F] * (scal * scal)
    b0 = a0_ref[...] * g
    out_ref[...] = jnp.dot(b0, wlin_ref[...], preferred_element_type=jnp.float32)


def _node_phase(a0, scal, cw, Wlin):
    return pl.pallas_call(
        _node_phase_body,
        grid=(N_PAD // NODE_BLK,),
        in_specs=[
            pl.BlockSpec((NODE_BLK, F), lambda i: (i, 0)),
            pl.BlockSpec((NODE_BLK, F), lambda i: (i, 0)),
            pl.BlockSpec((NODE_BLK, 3 * F), lambda i: (i, 0)),
            pl.BlockSpec((F, F), lambda i: (0, 0)),
        ],
        out_specs=pl.BlockSpec((NODE_BLK, F), lambda i: (i, 0)),
        out_shape=jax.ShapeDtypeStruct((N_PAD, F), jnp.float32),
    )(a0, scal, cw, Wlin)


def _sph(u):
    x, y, z = u[:, 0], u[:, 1], u[:, 2]
    s3 = float(np.sqrt(3.0)); s15 = float(np.sqrt(15.0)); s5 = float(np.sqrt(5.0))
    comps = [jnp.ones_like(x), s3 * x, s3 * y, s3 * z,
             s15 * x * y, s15 * y * z, 0.5 * s5 * (3.0 * z * z - 1.0),
             s15 * x * z, 0.5 * s15 * (x * x - y * y)]
    return jnp.stack(comps, axis=-1)


def _radial(r):
    n = jnp.arange(1, NB + 1, dtype=jnp.float32)
    rs = jnp.clip(r, 1e-9, None)
    rb = np.sqrt(2.0 / R_MAX) * jnp.sin(n * jnp.pi * rs / R_MAX) / rs
    x = r / R_MAX
    env = 1.0 - 21.0 * x ** 5 + 35.0 * x ** 6 - 15.0 * x ** 7
    env = jnp.where(x < 1.0, env, 0.0)
    return rb * env


def kernel(vectors, node_specie, senders, receivers, W_embed, W_up0, Wr1_0, Wr2_0, Wc0, Wlin0, Wro0, W_up1, Wr1_1, Wr2_1, Wc1, Wsc_lin1, Wsc_sp1, Wlin1, Wro1a, Wro1b):
    senders = senders.astype(jnp.int32)
    receivers = receivers.astype(jnp.int32)

    lengths = jnp.sqrt(jnp.sum(vectors * vectors, axis=-1, keepdims=True) + 1e-12)
    Y = _sph(vectors / lengths)
    ef = _radial(lengths)
    yr0 = Y * (jax.nn.silu(ef @ Wr1_0) @ Wr2_0)
    yr1 = Y * (jax.nn.silu(ef @ Wr1_1) @ Wr2_1)
    pad7 = jnp.zeros((E, L - SH), jnp.float32)
    yr0p = jnp.concatenate([yr0, pad7], axis=1).reshape(E // 8, 8 * L)
    yr1p = jnp.concatenate([yr1, pad7], axis=1).reshape(E // 8, 8 * L)

    emb = W_embed[node_specie]
    h0 = emb @ W_up0
    cw0 = Wc0[node_specie].reshape(N, 3 * F)
    cw1 = Wc1[node_specie].reshape(N, 3 * F)
    cw0p = jnp.concatenate([cw0, jnp.zeros((N_PAD - N, 3 * F), jnp.float32)], 0)
    cw1p = jnp.concatenate([cw1, jnp.zeros((N_PAD - N, 3 * F), jnp.float32)], 0)

    hist = _hist_kernel(receivers)
    offs, bstart, bcnt = _scan_kernel(hist)
    perm = _permute_kernel(receivers, senders, offs)

    def interaction(h, yrp, cwp, Wlin):
        hp = jnp.concatenate([h, jnp.zeros((N_PAD - N, F), jnp.float32)], 0)
        a0s = _edge_kernel(perm, bstart, bcnt, yrp, hp)
        a0, scal = a0s[:, :F], a0s[:, F:]
        return _node_phase(a0, scal, cwp, Wlin)[:N]

    nf1_0 = interaction(h0, yr0p, cw0p, Wlin0)
    ro0 = nf1_0 @ Wro0

    h1 = nf1_0 @ W_up1
    nf2_0 = interaction(h1, yr1p, cw1p, Wlin1)
    nf2_0 = nf2_0 + (nf1_0 @ Wsc_lin1) * Wsc_sp1[node_specie]
    ro1 = jax.nn.silu(nf2_0 @ Wro1a) @ Wro1b
    return jnp.stack([ro0, ro1], axis=1)
